# Initial kernel scaffold; baseline (speedup 1.0000x reference)
import jax
import jax.numpy as jnp
from jax.experimental import pallas as pl


def kernel(student_id, table, feat_age, feat_gender, feat_ethnicity, feat_location, feat_gpa, feat_test_scores, feat_courses, feat_major, feat_attendance, feat_participation, feat_feedback, feat_study_habits, feat_social_activity, feat_stress_level, W_age, W_gender, W_ethnicity, W_location, W_gpa, W_test_scores, W_courses, W_major, W_attendance, W_participation, W_feedback, W_study_habits, W_social_activity, W_stress_level, b_age, b_gender, b_ethnicity, b_location, b_gpa, b_test_scores, b_courses, b_major, b_attendance, b_participation, b_feedback, b_study_habits, b_social_activity, b_stress_level):
    raise NotImplementedError("write your pallas kernel here")


# speedup vs baseline: 1.5817x; 1.5817x over previous
import jax
import jax.numpy as jnp
from jax import lax
from jax.experimental import pallas as pl
from jax.experimental.pallas import tpu as pltpu
from jax.experimental.pallas import tpu_sc as plsc

_B = 16384
_D = 64
_NF = 14
_OUT_W = (_NF + 1) * _D
_NC = 2
_NS = 16
_NW = _NC * _NS
_RPW = _B // _NW
_C = 32
_NCH = _RPW // _C
_L = 16


_GATHER_DNUMS = lax.GatherDimensionNumbers(
    offset_dims=(), collapsed_slice_dims=(0,), start_index_map=(0,))


def _lane_bcast(v, lane):
    idx = jnp.full((_L, 1), lane, dtype=jnp.int32)
    return lax.gather(v, idx, _GATHER_DNUMS, (1,),
                      mode=lax.GatherScatterMode.PROMISE_IN_BOUNDS)


def _tower_body(idx_hbm, table_hbm, feats_hbm, w_hbm, b_hbm, out_hbm,
                idx_v, feats_v, w_v, b_v, rows_v, dense_v, gsem):
    c = lax.axis_index("c")
    s = lax.axis_index("s")
    wid = s * _NC + c
    base_row = wid * _RPW

    pltpu.sync_copy(feats_hbm.at[:, pl.ds(base_row, _RPW)], feats_v)
    pltpu.sync_copy(w_hbm, w_v)
    pltpu.sync_copy(b_hbm, b_v)

    def chunk(g, carry):
        rbase = base_row + g * _C
        pltpu.sync_copy(idx_hbm.at[wid * _NCH + g], idx_v)
        gcp = pltpu.async_copy(table_hbm.at[idx_v], rows_v, gsem)

        def group(t, carry2):
            fr = g * _C + t * _L
            for f in range(_NF):
                fv = feats_v[f, pl.ds(fr, _L)]
                bcs = [_lane_bcast(fv, r) for r in range(_L)]
                for q in range(_D // _L):
                    col = (f + 1) * _D + q * _L
                    wv = w_v[f, pl.ds(q * _L, _L)]
                    bv = b_v[f, pl.ds(q * _L, _L)]
                    for r in range(_L):
                        dense_v[t * _L + r, pl.ds(col, _L)] = bcs[r] * wv + bv
            return carry2

        lax.fori_loop(0, _C // _L, group, 0)
        gcp.wait()

        def basecopy(r, carry2):
            for q in range(_D // _L):
                dense_v[r, pl.ds(q * _L, _L)] = rows_v[r, pl.ds(q * _L, _L)]
            return carry2

        lax.fori_loop(0, _C, basecopy, 0)
        pltpu.sync_copy(dense_v, out_hbm.at[pl.ds(rbase, _C)])
        return carry

    lax.fori_loop(0, _NCH, chunk, 0)


@jax.jit
def _tower(idx2d, table, feats, w, b):
    kern = pl.kernel(
        _tower_body,
        out_type=jax.ShapeDtypeStruct((_B, _OUT_W), jnp.float32),
        mesh=plsc.VectorSubcoreMesh(core_axis_name="c", subcore_axis_name="s"),
        scratch_types=[
            pltpu.VMEM((_C,), jnp.int32),
            pltpu.VMEM((_NF, _RPW), jnp.float32),
            pltpu.VMEM((_NF, _D), jnp.float32),
            pltpu.VMEM((_NF, _D), jnp.float32),
            pltpu.VMEM((_C, 2 * _D), jnp.float32),
            pltpu.VMEM((_C, _OUT_W), jnp.float32),
            pltpu.SemaphoreType.DMA,
        ],
    )
    return kern(idx2d, table, feats, w, b)


def kernel(student_id, table,
           feat_age, feat_gender, feat_ethnicity, feat_location, feat_gpa,
           feat_test_scores, feat_courses, feat_major, feat_attendance,
           feat_participation, feat_feedback, feat_study_habits,
           feat_social_activity, feat_stress_level,
           W_age, W_gender, W_ethnicity, W_location, W_gpa,
           W_test_scores, W_courses, W_major, W_attendance,
           W_participation, W_feedback, W_study_habits,
           W_social_activity, W_stress_level,
           b_age, b_gender, b_ethnicity, b_location, b_gpa,
           b_test_scores, b_courses, b_major, b_attendance,
           b_participation, b_feedback, b_study_habits,
           b_social_activity, b_stress_level):
    feats = jnp.stack([
        feat_age, feat_gender, feat_ethnicity, feat_location, feat_gpa,
        feat_test_scores, feat_courses, feat_major, feat_attendance,
        feat_participation, feat_feedback, feat_study_habits,
        feat_social_activity, feat_stress_level])
    w = jnp.concatenate([
        W_age, W_gender, W_ethnicity, W_location, W_gpa,
        W_test_scores, W_courses, W_major, W_attendance,
        W_participation, W_feedback, W_study_habits,
        W_social_activity, W_stress_level], axis=0)
    b = jnp.stack([
        b_age, b_gender, b_ethnicity, b_location, b_gpa,
        b_test_scores, b_courses, b_major, b_attendance,
        b_participation, b_feedback, b_study_habits,
        b_social_activity, b_stress_level])
    idx2d = student_id.reshape(_B // _C, _C)
    table128 = jnp.concatenate(
        [table, jnp.zeros((table.shape[0], _D), table.dtype)], axis=1)
    return _tower(idx2d, table128, feats, w, b)


# speedup vs baseline: 1.8942x; 1.1976x over previous
import jax
import jax.numpy as jnp
from jax import lax
from jax.experimental import pallas as pl
from jax.experimental.pallas import tpu as pltpu
from jax.experimental.pallas import tpu_sc as plsc

_B = 16384
_D = 64
_NF = 14
_OUT_W = (_NF + 1) * _D
_NC = 2
_NS = 16
_NW = _NC * _NS
_RPW = _B // _NW
_C = 32
_NCH = _RPW // _C
_L = 16

_GATHER_DNUMS = lax.GatherDimensionNumbers(
    offset_dims=(), collapsed_slice_dims=(0,), start_index_map=(0,))


def _lane_bcast(v, lane):
    idx = jnp.full((_L, 1), lane, dtype=jnp.int32)
    return lax.gather(v, idx, _GATHER_DNUMS, (1,),
                      mode=lax.GatherScatterMode.PROMISE_IN_BOUNDS)


def _tower_body(idx_hbm, table_hbm, feats_hbm, w_hbm, b_hbm, out_hbm,
                idx_v, feats_v, w_v, b_v, rows_v, blk_v, gsem, osem):
    c = lax.axis_index("c")
    s = lax.axis_index("s")
    wid = s * _NC + c
    base_row = wid * _RPW

    pltpu.sync_copy(idx_hbm.at[pl.ds(wid * _NCH, _NCH)], idx_v)
    pltpu.sync_copy(feats_hbm.at[:, pl.ds(base_row, _RPW)], feats_v)
    pltpu.sync_copy(w_hbm, w_v)
    pltpu.sync_copy(b_hbm, b_v)

    def fire_gather(g, p):
        return pltpu.async_copy(table_hbm.at[idx_v.at[g]], rows_v.at[p],
                                gsem.at[p])

    def compute_dense(blk, g):
        def group(t, carry):
            fr = g * _C + t * _L
            def feat(f, carry2):
                fv = feats_v[f, pl.ds(fr, _L)]
                bcs = [_lane_bcast(fv, r) for r in range(_L)]
                for q in range(_D // _L):
                    wv = w_v[f, pl.ds(q * _L, _L)]
                    bv = b_v[f, pl.ds(q * _L, _L)]
                    col = (f + 1) * _D + q * _L
                    for r in range(_L):
                        blk[t * _L + r, pl.ds(col, _L)] = bcs[r] * wv + bv
                return carry2
            return lax.fori_loop(0, _NF, feat, carry)
        lax.fori_loop(0, _C // _L, group, 0)

    def base_into(blk, p):
        def row(r, carry):
            for q in range(_D // _L):
                blk[r, pl.ds(q * _L, _L)] = rows_v[p, r, pl.ds(q * _L, _L)]
            return carry
        lax.fori_loop(0, _C, row, 0)

    fire_gather(0, 0)

    def pair(h, carry):
        for p in range(2):
            g = h * 2 + p
            blk = blk_v.at[p]

            @pl.when(g + 1 < _NCH)
            def _():
                fire_gather(g + 1, 1 - p)

            @pl.when(g >= 2)
            def _():
                pltpu.make_async_copy(
                    blk, out_hbm.at[pl.ds(base_row + (g - 2) * _C, _C)],
                    osem.at[p]).wait()

            compute_dense(blk, g)
            pltpu.make_async_copy(table_hbm.at[idx_v.at[g]], rows_v.at[p],
                                  gsem.at[p]).wait()
            base_into(blk, p)
            pltpu.async_copy(blk, out_hbm.at[pl.ds(base_row + g * _C, _C)],
                             osem.at[p])
        return carry

    lax.fori_loop(0, _NCH // 2, pair, 0)

    for p in range(2):
        g = _NCH - 2 + p
        pltpu.make_async_copy(
            blk_v.at[p], out_hbm.at[pl.ds(base_row + g * _C, _C)],
            osem.at[p]).wait()


@jax.jit
def _tower(idx2d, table, feats, w, b):
    kern = pl.kernel(
        _tower_body,
        out_type=jax.ShapeDtypeStruct((_B, _OUT_W), jnp.float32),
        mesh=plsc.VectorSubcoreMesh(core_axis_name="c", subcore_axis_name="s"),
        scratch_types=[
            pltpu.VMEM((_NCH, _C), jnp.int32),
            pltpu.VMEM((_NF, _RPW), jnp.float32),
            pltpu.VMEM((_NF, _D), jnp.float32),
            pltpu.VMEM((_NF, _D), jnp.float32),
            pltpu.VMEM((2, _C, 2 * _D), jnp.float32),
            pltpu.VMEM((2, _C, _OUT_W), jnp.float32),
            pltpu.SemaphoreType.DMA((2,)),
            pltpu.SemaphoreType.DMA((2,)),
        ],
    )
    return kern(idx2d, table, feats, w, b)


def kernel(student_id, table,
           feat_age, feat_gender, feat_ethnicity, feat_location, feat_gpa,
           feat_test_scores, feat_courses, feat_major, feat_attendance,
           feat_participation, feat_feedback, feat_study_habits,
           feat_social_activity, feat_stress_level,
           W_age, W_gender, W_ethnicity, W_location, W_gpa,
           W_test_scores, W_courses, W_major, W_attendance,
           W_participation, W_feedback, W_study_habits,
           W_social_activity, W_stress_level,
           b_age, b_gender, b_ethnicity, b_location, b_gpa,
           b_test_scores, b_courses, b_major, b_attendance,
           b_participation, b_feedback, b_study_habits,
           b_social_activity, b_stress_level):
    feats = jnp.stack([
        feat_age, feat_gender, feat_ethnicity, feat_location, feat_gpa,
        feat_test_scores, feat_courses, feat_major, feat_attendance,
        feat_participation, feat_feedback, feat_study_habits,
        feat_social_activity, feat_stress_level])
    w = jnp.concatenate([
        W_age, W_gender, W_ethnicity, W_location, W_gpa,
        W_test_scores, W_courses, W_major, W_attendance,
        W_participation, W_feedback, W_study_habits,
        W_social_activity, W_stress_level], axis=0)
    b = jnp.stack([
        b_age, b_gender, b_ethnicity, b_location, b_gpa,
        b_test_scores, b_courses, b_major, b_attendance,
        b_participation, b_feedback, b_study_habits,
        b_social_activity, b_stress_level])
    idx2d = student_id.reshape(_B // _C, _C)
    table128 = jnp.concatenate(
        [table, jnp.zeros((table.shape[0], _D), table.dtype)], axis=1)
    return _tower(idx2d, table128, feats, w, b)


# speedup vs baseline: 2.0887x; 1.1027x over previous
import jax
import jax.numpy as jnp
from jax import lax
from jax.experimental import pallas as pl
from jax.experimental.pallas import tpu as pltpu
from jax.experimental.pallas import tpu_sc as plsc

_B = 16384
_D = 64
_NF = 14
_OUT_W = (_NF + 1) * _D
_NC = 2
_NS = 16
_NW = _NC * _NS
_RPW = _B // _NW
_C = 32
_NCH = _RPW // _C
_L = 16

_GATHER_DNUMS = lax.GatherDimensionNumbers(
    offset_dims=(), collapsed_slice_dims=(0,), start_index_map=(0,))


def _lane_bcast(v, lane):
    idx = jnp.full((_L, 1), lane, dtype=jnp.int32)
    return lax.gather(v, idx, _GATHER_DNUMS, (1,),
                      mode=lax.GatherScatterMode.PROMISE_IN_BOUNDS)


def _tower_body(*refs):
    sid_hbm = refs[0]
    table_hbm = refs[1]
    feat_hbms = refs[2:2 + _NF]
    w_hbms = refs[2 + _NF:2 + 2 * _NF]
    b_hbms = refs[2 + 2 * _NF:2 + 3 * _NF]
    out_hbm = refs[2 + 3 * _NF]
    (idx_v, feats_v, w_v, b_v, rows_v, blk_v, psem, gsem, osem) = \
        refs[3 + 3 * _NF:]

    c = lax.axis_index("c")
    s = lax.axis_index("s")
    wid = s * _NC + c
    base_row = wid * _RPW

    pro = [pltpu.async_copy(sid_hbm.at[pl.ds(base_row, _RPW)], idx_v, psem)]
    for f in range(_NF):
        pro.append(pltpu.async_copy(
            feat_hbms[f].at[pl.ds(base_row, _RPW)],
            feats_v.at[f], psem))
        pro.append(pltpu.async_copy(w_hbms[f], w_v.at[pl.ds(f, 1)], psem))
        pro.append(pltpu.async_copy(b_hbms[f], b_v.at[f], psem))
    for cp in pro:
        cp.wait()

    def fire_gather(g, p):
        return pltpu.async_copy(table_hbm.at[idx_v.at[pl.ds(g * _C, _C)]],
                                rows_v.at[p], gsem.at[p])

    def compute_dense(blk, g):
        def group(t, carry):
            fr = g * _C + t * _L
            def feat(f, carry2):
                fv = feats_v[f, pl.ds(fr, _L)]
                bcs = [_lane_bcast(fv, r) for r in range(_L)]
                for q in range(_D // _L):
                    wv = w_v[f, pl.ds(q * _L, _L)]
                    bv = b_v[f, pl.ds(q * _L, _L)]
                    col = (f + 1) * _D + q * _L
                    for r in range(_L):
                        blk[t * _L + r, pl.ds(col, _L)] = bcs[r] * wv + bv
                return carry2
            return lax.fori_loop(0, _NF, feat, carry)
        lax.fori_loop(0, _C // _L, group, 0)

    def base_into(blk, p):
        def row(r, carry):
            for q in range(_D // _L):
                blk[r, pl.ds(q * _L, _L)] = rows_v[p, r, pl.ds(q * _L, _L)]
            return carry
        lax.fori_loop(0, _C, row, 0)

    fire_gather(0, 0)

    def pair(h, carry):
        for p in range(2):
            g = h * 2 + p
            blk = blk_v.at[p]

            @pl.when(g + 1 < _NCH)
            def _():
                fire_gather(g + 1, 1 - p)

            @pl.when(g >= 2)
            def _():
                pltpu.make_async_copy(
                    blk, out_hbm.at[pl.ds(base_row + (g - 2) * _C, _C)],
                    osem.at[p]).wait()

            compute_dense(blk, g)
            pltpu.make_async_copy(
                table_hbm.at[idx_v.at[pl.ds(g * _C, _C)]], rows_v.at[p],
                gsem.at[p]).wait()
            base_into(blk, p)
            pltpu.async_copy(blk, out_hbm.at[pl.ds(base_row + g * _C, _C)],
                             osem.at[p])
        return carry

    lax.fori_loop(0, _NCH // 2, pair, 0)

    for p in range(2):
        g = _NCH - 2 + p
        pltpu.make_async_copy(
            blk_v.at[p], out_hbm.at[pl.ds(base_row + g * _C, _C)],
            osem.at[p]).wait()


@jax.jit
def _tower(sid, table128, *fwb):
    kern = pl.kernel(
        _tower_body,
        out_type=jax.ShapeDtypeStruct((_B, _OUT_W), jnp.float32),
        mesh=plsc.VectorSubcoreMesh(core_axis_name="c", subcore_axis_name="s"),
        scratch_types=[
            pltpu.VMEM((_RPW,), jnp.int32),
            pltpu.VMEM((_NF, _RPW), jnp.float32),
            pltpu.VMEM((_NF, _D), jnp.float32),
            pltpu.VMEM((_NF, _D), jnp.float32),
            pltpu.VMEM((2, _C, 2 * _D), jnp.float32),
            pltpu.VMEM((2, _C, _OUT_W), jnp.float32),
            pltpu.SemaphoreType.DMA,
            pltpu.SemaphoreType.DMA((2,)),
            pltpu.SemaphoreType.DMA((2,)),
        ],
    )
    return kern(sid, table128, *fwb)


def kernel(student_id, table,
           feat_age, feat_gender, feat_ethnicity, feat_location, feat_gpa,
           feat_test_scores, feat_courses, feat_major, feat_attendance,
           feat_participation, feat_feedback, feat_study_habits,
           feat_social_activity, feat_stress_level,
           W_age, W_gender, W_ethnicity, W_location, W_gpa,
           W_test_scores, W_courses, W_major, W_attendance,
           W_participation, W_feedback, W_study_habits,
           W_social_activity, W_stress_level,
           b_age, b_gender, b_ethnicity, b_location, b_gpa,
           b_test_scores, b_courses, b_major, b_attendance,
           b_participation, b_feedback, b_study_habits,
           b_social_activity, b_stress_level):
    table128 = jnp.concatenate(
        [table, jnp.zeros((table.shape[0], _D), table.dtype)], axis=1)
    return _tower(
        student_id, table128,
        feat_age, feat_gender, feat_ethnicity, feat_location, feat_gpa,
        feat_test_scores, feat_courses, feat_major, feat_attendance,
        feat_participation, feat_feedback, feat_study_habits,
        feat_social_activity, feat_stress_level,
        W_age, W_gender, W_ethnicity, W_location, W_gpa,
        W_test_scores, W_courses, W_major, W_attendance,
        W_participation, W_feedback, W_study_habits,
        W_social_activity, W_stress_level,
        b_age, b_gender, b_ethnicity, b_location, b_gpa,
        b_test_scores, b_courses, b_major, b_attendance,
        b_participation, b_feedback, b_study_habits,
        b_social_activity, b_stress_level)


# speedup vs baseline: 2.9068x; 1.3917x over previous
import jax
import jax.numpy as jnp
from jax import lax
from jax.experimental import pallas as pl
from jax.experimental.pallas import tpu as pltpu
from jax.experimental.pallas import tpu_sc as plsc

_B = 16384
_D = 64
_NF = 14
_OUT_W = (_NF + 1) * _D
_NW = 32
_CB = 128
_NITEM = (_NF + 1) * (_B // _CB) // _NW
_L = 16

_GATHER_DNUMS = lax.GatherDimensionNumbers(
    offset_dims=(), collapsed_slice_dims=(0,), start_index_map=(0,))


def _lane_bcast(v, lane):
    if isinstance(lane, int):
        idx = jnp.full((_L, 1), lane, dtype=jnp.int32)
    else:
        idx = jnp.broadcast_to(lane, (_L,)).astype(jnp.int32)[:, None]
    return lax.gather(v, idx, _GATHER_DNUMS, (1,),
                      mode=lax.GatherScatterMode.PROMISE_IN_BOUNDS)


def _tower_body(sid_hbm, table_hbm, feats_hbm, w_hbm, b_hbm, out_hbm,
                idx_v, w_v, b_v, gbuf, fbuf, blk_v, psem, gsem, fsem, osem):
    c_ax = lax.axis_index("c")
    s_ax = lax.axis_index("s")
    w = s_ax * 2 + c_ax

    def chunk_col(i):
        return w + 32 * lax.rem(i, 4)

    pro = []
    for j in range(4):
        cj = w + 32 * j
        pro.append(pltpu.async_copy(sid_hbm.at[pl.ds(cj * _CB, _CB)],
                                    idx_v.at[j], psem))
    pro.append(pltpu.async_copy(w_hbm, w_v, psem))
    pro.append(pltpu.async_copy(b_hbm, b_v, psem))
    for cp in pro:
        cp.wait()

    def fire_input(j, jp):
        @pl.when(j < 4)
        def _():
            pltpu.async_copy(table_hbm.at[idx_v.at[j]], gbuf.at[jp],
                             gsem.at[jp])

        @pl.when(j >= 4)
        def _():
            f = lax.div(j, 4) - 1
            pltpu.async_copy(
                feats_hbm.at[f, pl.ds(chunk_col(j) * _CB, _CB)],
                fbuf.at[jp], fsem.at[jp])

    def out_slice(i):
        return out_hbm.at[pl.ds(lax.div(i, 4) * _D, _D),
                          pl.ds(chunk_col(i) * _CB, _CB)]

    fire_input(0, 0)

    def item(i, carry):
        p = lax.rem(i, 2)
        blk = blk_v.at[p]

        @pl.when(i + 1 < _NITEM)
        def _():
            fire_input(i + 1, 1 - p)

        @pl.when(i >= 2)
        def _():
            pltpu.make_async_copy(blk, out_slice(i - 2), osem.at[p]).wait()

        @pl.when(i < 4)
        def _():
            pltpu.make_async_copy(table_hbm.at[idx_v.at[i]], gbuf.at[p],
                                  gsem.at[p]).wait()
            iota = lax.iota(jnp.int32, _L)

            def row(r, carry2):
                cols = jnp.broadcast_to(r, (_L,)).astype(jnp.int32)
                for q in range(_D // _L):
                    val = gbuf[p, r, pl.ds(q * _L, _L)]
                    plsc.store_scatter(blk, [iota + (q * _L), cols], val)
                return carry2

            lax.fori_loop(0, _CB, row, 0)

        @pl.when(i >= 4)
        def _():
            f = lax.div(i, 4) - 1
            pltpu.make_async_copy(
                feats_hbm.at[f, pl.ds(chunk_col(i) * _CB, _CB)],
                fbuf.at[p], fsem.at[p]).wait()
            fvs = [fbuf[p, pl.ds(rc * _L, _L)] for rc in range(_CB // _L)]

            def qloop(q, carry2):
                wv = w_v[f, pl.ds(q * _L, _L)]
                bv = b_v[f, pl.ds(q * _L, _L)]
                for d2 in range(_L):
                    wbc = _lane_bcast(wv, d2)
                    bbc = _lane_bcast(bv, d2)
                    drow = q * _L + d2
                    for rc in range(_CB // _L):
                        blk[drow, pl.ds(rc * _L, _L)] = wbc * fvs[rc] + bbc
                return carry2

            lax.fori_loop(0, _D // _L, qloop, 0)

        pltpu.async_copy(blk, out_slice(i), osem.at[p])
        return carry

    lax.fori_loop(0, _NITEM, item, 0)

    for j in (_NITEM - 2, _NITEM - 1):
        pltpu.make_async_copy(blk_v.at[j % 2], out_slice(j),
                              osem.at[j % 2]).wait()


@jax.jit
def _tower(sid, table128, feats, wmat, bmat):
    kern = pl.kernel(
        _tower_body,
        out_type=jax.ShapeDtypeStruct((_OUT_W, _B), jnp.float32),
        mesh=plsc.VectorSubcoreMesh(core_axis_name="c", subcore_axis_name="s"),
        compiler_params=pltpu.CompilerParams(needs_layout_passes=False),
        scratch_types=[
            pltpu.VMEM((4, _CB), jnp.int32),
            pltpu.VMEM((_NF, _D), jnp.float32),
            pltpu.VMEM((_NF, _D), jnp.float32),
            pltpu.VMEM((2, _CB, 2 * _D), jnp.float32),
            pltpu.VMEM((2, _CB), jnp.float32),
            pltpu.VMEM((2, _D, _CB), jnp.float32),
            pltpu.SemaphoreType.DMA,
            pltpu.SemaphoreType.DMA((2,)),
            pltpu.SemaphoreType.DMA((2,)),
            pltpu.SemaphoreType.DMA((2,)),
        ],
    )
    return kern(sid, table128, feats, wmat, bmat)


def kernel(student_id, table,
           feat_age, feat_gender, feat_ethnicity, feat_location, feat_gpa,
           feat_test_scores, feat_courses, feat_major, feat_attendance,
           feat_participation, feat_feedback, feat_study_habits,
           feat_social_activity, feat_stress_level,
           W_age, W_gender, W_ethnicity, W_location, W_gpa,
           W_test_scores, W_courses, W_major, W_attendance,
           W_participation, W_feedback, W_study_habits,
           W_social_activity, W_stress_level,
           b_age, b_gender, b_ethnicity, b_location, b_gpa,
           b_test_scores, b_courses, b_major, b_attendance,
           b_participation, b_feedback, b_study_habits,
           b_social_activity, b_stress_level):
    feats = jnp.stack([
        feat_age, feat_gender, feat_ethnicity, feat_location, feat_gpa,
        feat_test_scores, feat_courses, feat_major, feat_attendance,
        feat_participation, feat_feedback, feat_study_habits,
        feat_social_activity, feat_stress_level])
    wmat = jnp.concatenate([
        W_age, W_gender, W_ethnicity, W_location, W_gpa,
        W_test_scores, W_courses, W_major, W_attendance,
        W_participation, W_feedback, W_study_habits,
        W_social_activity, W_stress_level], axis=0)
    bmat = jnp.stack([
        b_age, b_gender, b_ethnicity, b_location, b_gpa,
        b_test_scores, b_courses, b_major, b_attendance,
        b_participation, b_feedback, b_study_habits,
        b_social_activity, b_stress_level])
    table128 = jnp.concatenate(
        [table, jnp.zeros((table.shape[0], _D), table.dtype)], axis=1)
    out2 = _tower(student_id, table128, feats, wmat, bmat)
    return out2.T
